# Initial kernel scaffold; baseline (speedup 1.0000x reference)
import functools

import jax
import jax.numpy as jnp
from jax import lax
from jax.experimental import pallas as pl
from jax.experimental.pallas import tpu as pltpu
from jax.experimental.pallas import tpu_sc as plsc

N = 10000
E = 320000
D = 128

NC = 2
NS = 16
NW = NC * NS
G = 125
EPW = E // NW
NCH = EPW // G
ROWS_PER_TILE = N // NS

_MESH = plsc.VectorSubcoreMesh(core_axis_name="c", subcore_axis_name="s")


def _fill_vmem_2d(ref, rows, cols, value):
    vec = jnp.full((16,), value, jnp.float32)

    def row_body(r, _):
        def col_body(j, __):
            ref[r, pl.ds(j * 16, 16)] = vec
            return 0

        return lax.fori_loop(0, cols // 16, col_body, 0)

    lax.fori_loop(0, rows, row_body, 0)


@functools.partial(
    pl.kernel,
    out_type=jax.ShapeDtypeStruct((NC, N, D), jnp.float32),
    mesh=_MESH,
    scratch_types=[
        pltpu.VMEM((NCH, G), jnp.int32),
        pltpu.VMEM((NCH, G), jnp.int32),
        pltpu.VMEM((G, D), jnp.float32),
        pltpu.VMEM_SHARED((N, D), jnp.float32),
        pltpu.SemaphoreType.DMA,
    ],
)
def _sc_agg(h_hbm, src_hbm, dst_hbm, out_hbm, src_v, dst_v, rows_v, acc_sh, sem):
    c = lax.axis_index("c")
    s = lax.axis_index("s")
    wid = c * NS + s

    _fill_vmem_2d(rows_v, G, D, 0.0)
    base_row = s * ROWS_PER_TILE
    for k in range(ROWS_PER_TILE // G):
        pltpu.sync_copy(rows_v, acc_sh.at[pl.ds(base_row + k * G, G)])

    pltpu.sync_copy(src_hbm.at[pl.ds(wid * NCH, NCH)], src_v)
    pltpu.sync_copy(dst_hbm.at[pl.ds(wid * NCH, NCH)], dst_v)

    plsc.subcore_barrier()

    def body(j, _):
        pltpu.async_copy(h_hbm.at[src_v.at[j]], rows_v, sem).wait()
        pltpu.sync_copy(rows_v, acc_sh.at[dst_v.at[j]], add=True)
        return 0

    lax.fori_loop(0, NCH, body, 0)

    plsc.subcore_barrier()

    pltpu.sync_copy(
        acc_sh.at[pl.ds(base_row, ROWS_PER_TILE)],
        out_hbm.at[c, pl.ds(base_row, ROWS_PER_TILE)],
    )


@functools.partial(
    pl.kernel,
    out_type=jax.ShapeDtypeStruct((NC, N, 16), jnp.float32),
    mesh=_MESH,
    scratch_types=[
        pltpu.VMEM((NCH, G), jnp.int32),
        pltpu.VMEM((G, 16), jnp.float32),
        pltpu.VMEM((G, 16), jnp.float32),
        pltpu.VMEM_SHARED((N, 16), jnp.float32),
    ],
)
def _sc_deg(dst_hbm, out_hbm, dst_v, ones_v, zero_v, acc_sh):
    c = lax.axis_index("c")
    s = lax.axis_index("s")
    wid = c * NS + s

    _fill_vmem_2d(zero_v, G, 16, 0.0)
    _fill_vmem_2d(ones_v, G, 16, 1.0)
    base_row = s * ROWS_PER_TILE
    for k in range(ROWS_PER_TILE // G):
        pltpu.sync_copy(zero_v, acc_sh.at[pl.ds(base_row + k * G, G)])

    pltpu.sync_copy(dst_hbm.at[pl.ds(wid * NCH, NCH)], dst_v)

    plsc.subcore_barrier()

    def body(j, _):
        pltpu.sync_copy(ones_v, acc_sh.at[dst_v.at[j]], add=True)
        return 0

    lax.fori_loop(0, NCH, body, 0)

    plsc.subcore_barrier()

    pltpu.sync_copy(
        acc_sh.at[pl.ds(base_row, ROWS_PER_TILE)],
        out_hbm.at[c, pl.ds(base_row, ROWS_PER_TILE)],
    )


BR = 1000


def _combine_body(h_ref, p0_ref, p1_ref, d0_ref, d1_ref, ws_ref, wn_ref, b_ref,
                  o_ref, *, relu):
    deg = jnp.maximum(d0_ref[:, 0:1] + d1_ref[:, 0:1], 1.0)
    agg = (p0_ref[...] + p1_ref[...]) / deg
    out = (
        jnp.dot(h_ref[...], ws_ref[...], preferred_element_type=jnp.float32)
        + jnp.dot(agg, wn_ref[...], preferred_element_type=jnp.float32)
        + b_ref[...]
    )
    if relu:
        out = jnp.maximum(out, 0.0)
    o_ref[...] = out


def _tc_combine(h, p0, p1, d0, d1, Ws, Wn, b, relu):
    return pl.pallas_call(
        functools.partial(_combine_body, relu=relu),
        grid=(N // BR,),
        in_specs=[
            pl.BlockSpec((BR, D), lambda i: (i, 0)),
            pl.BlockSpec((BR, D), lambda i: (i, 0)),
            pl.BlockSpec((BR, D), lambda i: (i, 0)),
            pl.BlockSpec((BR, 16), lambda i: (i, 0)),
            pl.BlockSpec((BR, 16), lambda i: (i, 0)),
            pl.BlockSpec((D, D), lambda i: (0, 0)),
            pl.BlockSpec((D, D), lambda i: (0, 0)),
            pl.BlockSpec((1, D), lambda i: (0, 0)),
        ],
        out_specs=pl.BlockSpec((BR, D), lambda i: (i, 0)),
        out_shape=jax.ShapeDtypeStruct((N, D), jnp.float32),
    )(h, p0, p1, d0, d1, Ws, Wn, b.reshape(1, D))


def kernel(x, edge_index, W_self0, W_neigh0, b0, W_self1, W_neigh1, b1,
           W_self2, W_neigh2, b2):
    src = edge_index[0].reshape(E // G, G)
    dst = edge_index[1].reshape(E // G, G)

    degp = _sc_deg(dst)
    d0, d1 = degp[0], degp[1]

    params = [
        (W_self0, W_neigh0, b0, True),
        (W_self1, W_neigh1, b1, True),
        (W_self2, W_neigh2, b2, False),
    ]
    h = x
    for Ws, Wn, b, relu in params:
        p = _sc_agg(h, src, dst)
        h = _tc_combine(h, p[0], p[1], d0, d1, Ws, Wn, b, relu)
    return h


# speedup vs baseline: 7.7607x; 7.7607x over previous
import functools

import jax
import jax.numpy as jnp
from jax import lax
from jax.experimental import pallas as pl
from jax.experimental.pallas import tpu as pltpu
from jax.experimental.pallas import tpu_sc as plsc

N = 10000
E = 320000
D = 128

NC = 2
NS = 16
NW = NC * NS
G = 128
NP = 10240
EP = NW * 80 * G
NCH = EP // NW // G
ROWS_PER_TILE = NP // NS

_MESH = plsc.VectorSubcoreMesh(core_axis_name="c", subcore_axis_name="s")


def _fill_vmem_2d(ref, rows, cols, value):
    vec = jnp.full((16,), value, jnp.float32)

    def row_body(r, _):
        def col_body(j, __):
            ref[r, pl.ds(j * 16, 16)] = vec
            return 0

        return lax.fori_loop(0, cols // 16, col_body, 0)

    lax.fori_loop(0, rows, row_body, 0)


@functools.partial(
    pl.kernel,
    out_type=jax.ShapeDtypeStruct((NC, NP, D), jnp.float32),
    mesh=_MESH,
    scratch_types=[
        pltpu.VMEM((NCH, G), jnp.int32),
        pltpu.VMEM((NCH, G), jnp.int32),
        pltpu.VMEM((G, D), jnp.float32),
        pltpu.VMEM_SHARED((NP, D), jnp.float32),
        pltpu.SemaphoreType.DMA,
    ],
)
def _sc_agg(h_hbm, src_hbm, dst_hbm, out_hbm, src_v, dst_v, rows_v, acc_sh, sem):
    c = lax.axis_index("c")
    s = lax.axis_index("s")
    wid = c * NS + s

    _fill_vmem_2d(rows_v, G, D, 0.0)
    base_row = s * ROWS_PER_TILE
    for k in range(ROWS_PER_TILE // G):
        pltpu.sync_copy(rows_v, acc_sh.at[pl.ds(base_row + k * G, G)])

    pltpu.sync_copy(src_hbm.at[pl.ds(wid * NCH, NCH)], src_v)
    pltpu.sync_copy(dst_hbm.at[pl.ds(wid * NCH, NCH)], dst_v)

    plsc.subcore_barrier()

    def body(j, _):
        pltpu.async_copy(h_hbm.at[src_v.at[j]], rows_v, sem).wait()
        pltpu.sync_copy(rows_v, acc_sh.at[dst_v.at[j]], add=True)
        return 0

    lax.fori_loop(0, NCH, body, 0)

    plsc.subcore_barrier()

    pltpu.sync_copy(
        acc_sh.at[pl.ds(base_row, ROWS_PER_TILE)],
        out_hbm.at[c, pl.ds(base_row, ROWS_PER_TILE)],
    )


@functools.partial(
    pl.kernel,
    out_type=jax.ShapeDtypeStruct((NC, NP, D), jnp.float32),
    mesh=_MESH,
    scratch_types=[
        pltpu.VMEM((NCH, G), jnp.int32),
        pltpu.VMEM((G, D), jnp.float32),
        pltpu.VMEM((G, D), jnp.float32),
        pltpu.VMEM_SHARED((NP, D), jnp.float32),
    ],
)
def _sc_deg(dst_hbm, out_hbm, dst_v, ones_v, zero_v, acc_sh):
    c = lax.axis_index("c")
    s = lax.axis_index("s")
    wid = c * NS + s

    _fill_vmem_2d(zero_v, G, D, 0.0)
    _fill_vmem_2d(ones_v, G, D, 1.0)
    base_row = s * ROWS_PER_TILE
    for k in range(ROWS_PER_TILE // G):
        pltpu.sync_copy(zero_v, acc_sh.at[pl.ds(base_row + k * G, G)])

    pltpu.sync_copy(dst_hbm.at[pl.ds(wid * NCH, NCH)], dst_v)

    plsc.subcore_barrier()

    def body(j, _):
        pltpu.sync_copy(ones_v, acc_sh.at[dst_v.at[j]], add=True)
        return 0

    lax.fori_loop(0, NCH, body, 0)

    plsc.subcore_barrier()

    pltpu.sync_copy(
        acc_sh.at[pl.ds(base_row, ROWS_PER_TILE)],
        out_hbm.at[c, pl.ds(base_row, ROWS_PER_TILE)],
    )


BR = 1000


def _combine_body(h_ref, p0_ref, p1_ref, d0_ref, d1_ref, ws_ref, wn_ref, b_ref,
                  o_ref, *, relu):
    deg = jnp.maximum(d0_ref[:, 0:1] + d1_ref[:, 0:1], 1.0)
    agg = (p0_ref[...] + p1_ref[...]) / deg
    out = (
        jnp.dot(h_ref[...], ws_ref[...], preferred_element_type=jnp.float32)
        + jnp.dot(agg, wn_ref[...], preferred_element_type=jnp.float32)
        + b_ref[...]
    )
    if relu:
        out = jnp.maximum(out, 0.0)
    o_ref[...] = out


def _tc_combine(h, p0, p1, d0, d1, Ws, Wn, b, relu):
    return pl.pallas_call(
        functools.partial(_combine_body, relu=relu),
        grid=(N // BR,),
        in_specs=[
            pl.BlockSpec((BR, D), lambda i: (i, 0)),
            pl.BlockSpec((BR, D), lambda i: (i, 0)),
            pl.BlockSpec((BR, D), lambda i: (i, 0)),
            pl.BlockSpec((BR, 16), lambda i: (i, 0)),
            pl.BlockSpec((BR, 16), lambda i: (i, 0)),
            pl.BlockSpec((D, D), lambda i: (0, 0)),
            pl.BlockSpec((D, D), lambda i: (0, 0)),
            pl.BlockSpec((1, D), lambda i: (0, 0)),
        ],
        out_specs=pl.BlockSpec((BR, D), lambda i: (i, 0)),
        out_shape=jax.ShapeDtypeStruct((N, D), jnp.float32),
    )(h, p0, p1, d0, d1, Ws, Wn, b.reshape(1, D))


def kernel(x, edge_index, W_self0, W_neigh0, b0, W_self1, W_neigh1, b1,
           W_self2, W_neigh2, b2):
    n_pad = EP - E
    pad_ids = jnp.arange(n_pad, dtype=jnp.int32)
    src = jnp.concatenate([edge_index[0], pad_ids % N]).reshape(EP // G, G)
    dst = jnp.concatenate([edge_index[1], N + pad_ids % (NP - N)]).reshape(
        EP // G, G)

    degp = _sc_deg(dst)
    d0 = degp[0, :N, :16]
    d1 = degp[1, :N, :16]

    params = [
        (W_self0, W_neigh0, b0, True),
        (W_self1, W_neigh1, b1, True),
        (W_self2, W_neigh2, b2, False),
    ]
    h = x
    for Ws, Wn, b, relu in params:
        p = _sc_agg(h, src, dst)
        h = _tc_combine(h, p[0, :N], p[1, :N], d0, d1, Ws, Wn, b, relu)
    return h


# speedup vs baseline: 9.5985x; 1.2368x over previous
import functools

import jax
import jax.numpy as jnp
from jax import lax
from jax.experimental import pallas as pl
from jax.experimental.pallas import tpu as pltpu
from jax.experimental.pallas import tpu_sc as plsc

N = 10000
E = 320000
D = 128

NC = 2
NS = 16
NW = NC * NS
G = 128
NP = 10240
EP = NW * 80 * G
NCH = EP // NW // G
ROWS_PER_TILE = NP // NS

_MESH = plsc.VectorSubcoreMesh(core_axis_name="c", subcore_axis_name="s")


def _fill_vmem_2d(ref, rows, cols, value):
    vec = jnp.full((16,), value, jnp.float32)

    def row_body(r, _):
        def col_body(j, __):
            ref[r, pl.ds(j * 16, 16)] = vec
            return 0

        return lax.fori_loop(0, cols // 16, col_body, 0)

    lax.fori_loop(0, rows, row_body, 0)


@functools.partial(
    pl.kernel,
    out_type=jax.ShapeDtypeStruct((NC, NP, D), jnp.float32),
    mesh=_MESH,
    scratch_types=[
        pltpu.VMEM((NCH // 2, G), jnp.int32),
        pltpu.VMEM((NCH // 2, G), jnp.int32),
        pltpu.VMEM((G, D), jnp.float32),
        pltpu.VMEM((G, D), jnp.float32),
        pltpu.VMEM_SHARED((NP, D), jnp.float32),
        pltpu.SemaphoreType.DMA,
        pltpu.SemaphoreType.DMA,
    ],
)
def _sc_agg(h_hbm, src_hbm, dst_hbm, out_hbm, src_v, dst_v, rows0, rows1,
            acc_sh, sem0, sem1):
    c = lax.axis_index("c")
    s = lax.axis_index("s")
    wid = c * NS + s
    HCH = NCH // 2

    _fill_vmem_2d(rows0, G, D, 0.0)
    base_row = s * ROWS_PER_TILE
    for k in range(ROWS_PER_TILE // G):
        pltpu.sync_copy(rows0, acc_sh.at[pl.ds(base_row + k * G, G)])

    plsc.subcore_barrier()

    for b in range(2):
        pltpu.sync_copy(src_hbm.at[pl.ds(wid * NCH + b * HCH, HCH)], src_v)
        pltpu.sync_copy(dst_hbm.at[pl.ds(wid * NCH + b * HCH, HCH)], dst_v)
        pltpu.async_copy(h_hbm.at[src_v.at[0]], rows0, sem0)

        def body(jj, _):
            j = jj * 2
            pltpu.make_async_copy(h_hbm.at[src_v.at[0]], rows0, sem0).wait()
            pltpu.async_copy(h_hbm.at[src_v.at[j + 1]], rows1, sem1)
            pltpu.sync_copy(rows0, acc_sh.at[dst_v.at[j]], add=True)
            pltpu.make_async_copy(h_hbm.at[src_v.at[0]], rows1, sem1).wait()
            pltpu.async_copy(h_hbm.at[src_v.at[(j + 2) % HCH]], rows0, sem0)
            pltpu.sync_copy(rows1, acc_sh.at[dst_v.at[j + 1]], add=True)
            return 0

        lax.fori_loop(0, HCH // 2, body, 0)
        pltpu.make_async_copy(h_hbm.at[src_v.at[0]], rows0, sem0).wait()

    plsc.subcore_barrier()

    pltpu.sync_copy(
        acc_sh.at[pl.ds(base_row, ROWS_PER_TILE)],
        out_hbm.at[c, pl.ds(base_row, ROWS_PER_TILE)],
    )


@functools.partial(
    pl.kernel,
    out_type=jax.ShapeDtypeStruct((NC, NP, D), jnp.float32),
    mesh=_MESH,
    scratch_types=[
        pltpu.VMEM((NCH, G), jnp.int32),
        pltpu.VMEM((G, D), jnp.float32),
        pltpu.VMEM((G, D), jnp.float32),
        pltpu.VMEM_SHARED((NP, D), jnp.float32),
    ],
)
def _sc_deg(dst_hbm, out_hbm, dst_v, ones_v, zero_v, acc_sh):
    c = lax.axis_index("c")
    s = lax.axis_index("s")
    wid = c * NS + s

    _fill_vmem_2d(zero_v, G, D, 0.0)
    _fill_vmem_2d(ones_v, G, D, 1.0)
    base_row = s * ROWS_PER_TILE
    for k in range(ROWS_PER_TILE // G):
        pltpu.sync_copy(zero_v, acc_sh.at[pl.ds(base_row + k * G, G)])

    pltpu.sync_copy(dst_hbm.at[pl.ds(wid * NCH, NCH)], dst_v)

    plsc.subcore_barrier()

    def body(j, _):
        pltpu.sync_copy(ones_v, acc_sh.at[dst_v.at[j]], add=True)
        return 0

    lax.fori_loop(0, NCH, body, 0)

    plsc.subcore_barrier()

    pltpu.sync_copy(
        acc_sh.at[pl.ds(base_row, ROWS_PER_TILE)],
        out_hbm.at[c, pl.ds(base_row, ROWS_PER_TILE)],
    )


BR = 1000


def _combine_body(h_ref, p0_ref, p1_ref, d0_ref, d1_ref, ws_ref, wn_ref, b_ref,
                  o_ref, *, relu):
    deg = jnp.maximum(d0_ref[:, 0:1] + d1_ref[:, 0:1], 1.0)
    agg = (p0_ref[...] + p1_ref[...]) / deg
    out = (
        jnp.dot(h_ref[...], ws_ref[...], preferred_element_type=jnp.float32)
        + jnp.dot(agg, wn_ref[...], preferred_element_type=jnp.float32)
        + b_ref[...]
    )
    if relu:
        out = jnp.maximum(out, 0.0)
    o_ref[...] = out


def _tc_combine(h, p0, p1, d0, d1, Ws, Wn, b, relu):
    return pl.pallas_call(
        functools.partial(_combine_body, relu=relu),
        grid=(N // BR,),
        in_specs=[
            pl.BlockSpec((BR, D), lambda i: (i, 0)),
            pl.BlockSpec((BR, D), lambda i: (i, 0)),
            pl.BlockSpec((BR, D), lambda i: (i, 0)),
            pl.BlockSpec((BR, 16), lambda i: (i, 0)),
            pl.BlockSpec((BR, 16), lambda i: (i, 0)),
            pl.BlockSpec((D, D), lambda i: (0, 0)),
            pl.BlockSpec((D, D), lambda i: (0, 0)),
            pl.BlockSpec((1, D), lambda i: (0, 0)),
        ],
        out_specs=pl.BlockSpec((BR, D), lambda i: (i, 0)),
        out_shape=jax.ShapeDtypeStruct((N, D), jnp.float32),
    )(h, p0, p1, d0, d1, Ws, Wn, b.reshape(1, D))


def kernel(x, edge_index, W_self0, W_neigh0, b0, W_self1, W_neigh1, b1,
           W_self2, W_neigh2, b2):
    n_pad = EP - E
    pad_ids = jnp.arange(n_pad, dtype=jnp.int32)
    src = jnp.concatenate([edge_index[0], pad_ids % N]).reshape(EP // G, G)
    dst = jnp.concatenate([edge_index[1], N + pad_ids % (NP - N)]).reshape(
        EP // G, G)

    degp = _sc_deg(dst)
    d0 = degp[0, :N, :16]
    d1 = degp[1, :N, :16]

    params = [
        (W_self0, W_neigh0, b0, True),
        (W_self1, W_neigh1, b1, True),
        (W_self2, W_neigh2, b2, False),
    ]
    h = x
    for Ws, Wn, b, relu in params:
        p = _sc_agg(h, src, dst)
        h = _tc_combine(h, p[0, :N], p[1, :N], d0, d1, Ws, Wn, b, relu)
    return h


# speedup vs baseline: 10.0453x; 1.0466x over previous
import functools

import jax
import jax.numpy as jnp
from jax import lax
from jax.experimental import pallas as pl
from jax.experimental.pallas import tpu as pltpu
from jax.experimental.pallas import tpu_sc as plsc

N = 10000
E = 320000
D = 128

NC = 2
NS = 16
NW = NC * NS
G = 128
NP = 10240
EP = NW * 80 * G
NCH = EP // NW // G
ROWS_PER_TILE = NP // NS

_MESH = plsc.VectorSubcoreMesh(core_axis_name="c", subcore_axis_name="s")


def _fill_vmem_2d(ref, rows, cols, value):
    vec = jnp.full((16,), value, jnp.float32)

    def row_body(r, _):
        def col_body(j, __):
            ref[r, pl.ds(j * 16, 16)] = vec
            return 0

        return lax.fori_loop(0, cols // 16, col_body, 0)

    lax.fori_loop(0, rows, row_body, 0)


@functools.partial(
    pl.kernel,
    out_type=jax.ShapeDtypeStruct((NC, NP, D), jnp.float32),
    mesh=_MESH,
    scratch_types=[
        pltpu.VMEM((NCH // 2, G), jnp.int32),
        pltpu.VMEM((NCH // 2, G), jnp.int32),
        pltpu.VMEM((G, D), jnp.float32),
        pltpu.VMEM((G, D), jnp.float32),
        pltpu.VMEM_SHARED((NP, D), jnp.float32),
        pltpu.SemaphoreType.DMA,
        pltpu.SemaphoreType.DMA,
    ],
)
def _sc_agg(h_hbm, src_hbm, dst_hbm, out_hbm, src_v, dst_v, rows0, rows1,
            acc_sh, sem0, sem1):
    c = lax.axis_index("c")
    s = lax.axis_index("s")
    wid = c * NS + s
    HCH = NCH // 2

    _fill_vmem_2d(rows0, G, D, 0.0)
    base_row = s * ROWS_PER_TILE
    for k in range(ROWS_PER_TILE // G):
        pltpu.sync_copy(rows0, acc_sh.at[pl.ds(base_row + k * G, G)])

    plsc.subcore_barrier()

    for b in range(2):
        pltpu.sync_copy(src_hbm.at[pl.ds(wid * NCH + b * HCH, HCH)], src_v)
        pltpu.sync_copy(dst_hbm.at[pl.ds(wid * NCH + b * HCH, HCH)], dst_v)
        pltpu.async_copy(h_hbm.at[src_v.at[0]], rows0, sem0)

        def body(jj, _):
            j = jj * 2
            pltpu.make_async_copy(h_hbm.at[src_v.at[0]], rows0, sem0).wait()
            pltpu.async_copy(h_hbm.at[src_v.at[j + 1]], rows1, sem1)
            pltpu.sync_copy(rows0, acc_sh.at[dst_v.at[j]], add=True)
            pltpu.make_async_copy(h_hbm.at[src_v.at[0]], rows1, sem1).wait()
            pltpu.async_copy(h_hbm.at[src_v.at[(j + 2) % HCH]], rows0, sem0)
            pltpu.sync_copy(rows1, acc_sh.at[dst_v.at[j + 1]], add=True)
            return 0

        lax.fori_loop(0, HCH // 2, body, 0)
        pltpu.make_async_copy(h_hbm.at[src_v.at[0]], rows0, sem0).wait()

    plsc.subcore_barrier()

    pltpu.sync_copy(
        acc_sh.at[pl.ds(base_row, ROWS_PER_TILE)],
        out_hbm.at[c, pl.ds(base_row, ROWS_PER_TILE)],
    )


@functools.partial(
    pl.kernel,
    out_type=jax.ShapeDtypeStruct((NC, NP, D), jnp.float32),
    mesh=_MESH,
    scratch_types=[
        pltpu.VMEM((NCH, G), jnp.int32),
        pltpu.VMEM((G, D), jnp.float32),
        pltpu.VMEM((G, D), jnp.float32),
        pltpu.VMEM_SHARED((NP, D), jnp.float32),
        pltpu.SemaphoreType.DMA,
    ],
)
def _sc_deg(dst_hbm, out_hbm, dst_v, ones_v, zero_v, acc_sh, sem):
    c = lax.axis_index("c")
    s = lax.axis_index("s")
    wid = c * NS + s

    _fill_vmem_2d(zero_v, G, D, 0.0)
    _fill_vmem_2d(ones_v, G, D, 1.0)
    base_row = s * ROWS_PER_TILE
    for k in range(ROWS_PER_TILE // G):
        pltpu.sync_copy(zero_v, acc_sh.at[pl.ds(base_row + k * G, G)])

    pltpu.sync_copy(dst_hbm.at[pl.ds(wid * NCH, NCH)], dst_v)

    plsc.subcore_barrier()

    QD = 8
    for k in range(QD):
        pltpu.async_copy(ones_v, acc_sh.at[dst_v.at[k]], sem, add=True)

    def body(j, _):
        pltpu.async_copy(ones_v, acc_sh.at[dst_v.at[j]], sem, add=True)
        pltpu.make_async_copy(ones_v, acc_sh.at[dst_v.at[0]], sem).wait()
        return 0

    lax.fori_loop(QD, NCH, body, 0)
    for k in range(QD):
        pltpu.make_async_copy(ones_v, acc_sh.at[dst_v.at[0]], sem).wait()

    plsc.subcore_barrier()

    pltpu.sync_copy(
        acc_sh.at[pl.ds(base_row, ROWS_PER_TILE)],
        out_hbm.at[c, pl.ds(base_row, ROWS_PER_TILE)],
    )


BR = 1000


def _combine_body(h_ref, p0_ref, p1_ref, d0_ref, d1_ref, ws_ref, wn_ref, b_ref,
                  o_ref, *, relu):
    deg = jnp.maximum(d0_ref[0, :, 0:1] + d1_ref[0, :, 0:1], 1.0)
    agg = (p0_ref[0] + p1_ref[0]) / deg
    out = (
        jnp.dot(h_ref[...], ws_ref[...], preferred_element_type=jnp.float32)
        + jnp.dot(agg, wn_ref[...], preferred_element_type=jnp.float32)
        + b_ref[...]
    )
    if relu:
        out = jnp.maximum(out, 0.0)
    o_ref[...] = out


def _tc_combine(h, p, degp, Ws, Wn, b, relu):
    return pl.pallas_call(
        functools.partial(_combine_body, relu=relu),
        grid=(N // BR,),
        in_specs=[
            pl.BlockSpec((BR, D), lambda i: (i, 0)),
            pl.BlockSpec((1, BR, D), lambda i: (0, i, 0)),
            pl.BlockSpec((1, BR, D), lambda i: (1, i, 0)),
            pl.BlockSpec((1, BR, D), lambda i: (0, i, 0)),
            pl.BlockSpec((1, BR, D), lambda i: (1, i, 0)),
            pl.BlockSpec((D, D), lambda i: (0, 0)),
            pl.BlockSpec((D, D), lambda i: (0, 0)),
            pl.BlockSpec((1, D), lambda i: (0, 0)),
        ],
        out_specs=pl.BlockSpec((BR, D), lambda i: (i, 0)),
        out_shape=jax.ShapeDtypeStruct((N, D), jnp.float32),
    )(h, p, p, degp, degp, Ws, Wn, b.reshape(1, D))


def kernel(x, edge_index, W_self0, W_neigh0, b0, W_self1, W_neigh1, b1,
           W_self2, W_neigh2, b2):
    n_pad = EP - E
    pad_ids = jnp.arange(n_pad, dtype=jnp.int32)
    src = jnp.concatenate([edge_index[0], pad_ids % N]).reshape(EP // G, G)
    dst = jnp.concatenate([edge_index[1], N + pad_ids % (NP - N)]).reshape(
        EP // G, G)

    degp = _sc_deg(dst)

    params = [
        (W_self0, W_neigh0, b0, True),
        (W_self1, W_neigh1, b1, True),
        (W_self2, W_neigh2, b2, False),
    ]
    h = x
    for Ws, Wn, b, relu in params:
        p = _sc_agg(h, src, dst)
        h = _tc_combine(h, p, degp, Ws, Wn, b, relu)
    return h


# speedup vs baseline: 10.6665x; 1.0618x over previous
import functools

import jax
import jax.numpy as jnp
from jax import lax
from jax.experimental import pallas as pl
from jax.experimental.pallas import tpu as pltpu
from jax.experimental.pallas import tpu_sc as plsc

N = 10000
E = 320000
D = 128

NC = 2
NS = 16
NW = NC * NS
G = 128
NP = 10240
EP = NW * 80 * G
NCH = EP // NW // G
ROWS_PER_TILE = NP // NS

_MESH = plsc.VectorSubcoreMesh(core_axis_name="c", subcore_axis_name="s")


def _fill_vmem_2d(ref, rows, cols, value):
    vec = jnp.full((16,), value, jnp.float32)

    def row_body(r, _):
        def col_body(j, __):
            ref[r, pl.ds(j * 16, 16)] = vec
            return 0

        return lax.fori_loop(0, cols // 16, col_body, 0)

    lax.fori_loop(0, rows, row_body, 0)


@functools.partial(
    pl.kernel,
    out_type=jax.ShapeDtypeStruct((NC, NP, D), jnp.float32),
    mesh=_MESH,
    scratch_types=[
        pltpu.VMEM((NCH // 2, G), jnp.int32),
        pltpu.VMEM((NCH // 2, G), jnp.int32),
        pltpu.VMEM((G, D), jnp.float32),
        pltpu.VMEM((G, D), jnp.float32),
        pltpu.VMEM_SHARED((NP, D), jnp.float32),
        pltpu.SemaphoreType.DMA,
        pltpu.SemaphoreType.DMA,
    ],
)
def _sc_agg(h_hbm, src_hbm, dst_hbm, out_hbm, src_v, dst_v, rows0, rows1,
            acc_sh, sem0, sem1):
    c = lax.axis_index("c")
    s = lax.axis_index("s")
    wid = c * NS + s
    HCH = NCH // 2

    _fill_vmem_2d(rows0, G, D, 0.0)
    base_row = s * ROWS_PER_TILE
    for k in range(ROWS_PER_TILE // G):
        pltpu.sync_copy(rows0, acc_sh.at[pl.ds(base_row + k * G, G)])

    plsc.subcore_barrier()

    for b in range(2):
        pltpu.sync_copy(src_hbm.at[pl.ds(wid * NCH + b * HCH, HCH)], src_v)
        pltpu.sync_copy(dst_hbm.at[pl.ds(wid * NCH + b * HCH, HCH)], dst_v)
        pltpu.async_copy(h_hbm.at[src_v.at[0]], rows0, sem0)

        def body(jj, _):
            j = jj * 2
            pltpu.make_async_copy(h_hbm.at[src_v.at[0]], rows0, sem0).wait()
            pltpu.async_copy(h_hbm.at[src_v.at[j + 1]], rows1, sem1)
            pltpu.sync_copy(rows0, acc_sh.at[dst_v.at[j]], add=True)
            pltpu.make_async_copy(h_hbm.at[src_v.at[0]], rows1, sem1).wait()
            pltpu.async_copy(h_hbm.at[src_v.at[(j + 2) % HCH]], rows0, sem0)
            pltpu.sync_copy(rows1, acc_sh.at[dst_v.at[j + 1]], add=True)
            return 0

        lax.fori_loop(0, HCH // 2, body, 0)
        pltpu.make_async_copy(h_hbm.at[src_v.at[0]], rows0, sem0).wait()

    plsc.subcore_barrier()

    pltpu.sync_copy(
        acc_sh.at[pl.ds(base_row, ROWS_PER_TILE)],
        out_hbm.at[c, pl.ds(base_row, ROWS_PER_TILE)],
    )


@functools.partial(
    pl.kernel,
    out_type=jax.ShapeDtypeStruct((NC, NP * D), jnp.float32),
    mesh=_MESH,
    compiler_params=pltpu.CompilerParams(needs_layout_passes=False),
    scratch_types=[
        pltpu.VMEM((NCH, G), jnp.int32),
        pltpu.VMEM((NP,), jnp.float32),
        pltpu.VMEM((NS, ROWS_PER_TILE), jnp.float32),
        pltpu.VMEM((ROWS_PER_TILE,), jnp.float32),
        pltpu.VMEM((G * D,), jnp.float32),
        pltpu.VMEM_SHARED((NS, NP), jnp.float32),
        pltpu.SemaphoreType.DMA,
    ],
)
def _sc_deg(dst_hbm, out_hbm, dst_v, hist, gath, red, brow, hists_sh, sem):
    c = lax.axis_index("c")
    s = lax.axis_index("s")
    wid = c * NS + s

    pltpu.sync_copy(dst_hbm.at[pl.ds(wid * NCH, NCH)], dst_v)

    zeros16 = jnp.zeros((16,), jnp.float32)

    def zb(i, _):
        hist[pl.ds(i * 16, 16)] = zeros16
        return 0

    lax.fori_loop(0, NP // 16, zb, 0)

    ones16 = jnp.ones((16,), jnp.float32)

    def hb(kk, _):
        idx = dst_v[kk // 8, pl.ds(lax.rem(kk, 8) * 16, 16)]
        plsc.addupdate_scatter(hist, [idx], ones16)
        return 0

    lax.fori_loop(0, NCH * 8, hb, 0)

    pltpu.sync_copy(hist, hists_sh.at[s])
    plsc.subcore_barrier()

    base = s * ROWS_PER_TILE
    pltpu.sync_copy(hists_sh.at[:, pl.ds(base, ROWS_PER_TILE)], gath)

    def rb(v, _):
        def rr(r, a):
            return a + gath[r, pl.ds(v * 16, 16)]

        red[pl.ds(v * 16, 16)] = lax.fori_loop(1, NS, rr,
                                               gath[0, pl.ds(v * 16, 16)])
        return 0

    lax.fori_loop(0, ROWS_PER_TILE // 16, rb, 0)

    for nb in range(ROWS_PER_TILE // G):
        def bb(n, _):
            val = plsc.load_gather(
                red, [jnp.full((16,), nb * G + n, jnp.int32)])

            def cb(q, _):
                brow[pl.ds(n * D + q * 16, 16)] = val
                return 0

            return lax.fori_loop(0, D // 16, cb, 0)

        lax.fori_loop(0, G, bb, 0)
        pltpu.sync_copy(brow,
                        out_hbm.at[c, pl.ds((base + nb * G) * D, G * D)])


BR = 1000


def _combine_body(h_ref, p0_ref, p1_ref, d0_ref, d1_ref, ws_ref, wn_ref, b_ref,
                  o_ref, *, relu):
    deg = jnp.maximum(d0_ref[0, :, 0:1] + d1_ref[0, :, 0:1], 1.0)
    agg = (p0_ref[0] + p1_ref[0]) / deg
    out = (
        jnp.dot(h_ref[...], ws_ref[...], preferred_element_type=jnp.float32)
        + jnp.dot(agg, wn_ref[...], preferred_element_type=jnp.float32)
        + b_ref[...]
    )
    if relu:
        out = jnp.maximum(out, 0.0)
    o_ref[...] = out


def _tc_combine(h, p, degp, Ws, Wn, b, relu):
    return pl.pallas_call(
        functools.partial(_combine_body, relu=relu),
        grid=(N // BR,),
        in_specs=[
            pl.BlockSpec((BR, D), lambda i: (i, 0)),
            pl.BlockSpec((1, BR, D), lambda i: (0, i, 0)),
            pl.BlockSpec((1, BR, D), lambda i: (1, i, 0)),
            pl.BlockSpec((1, BR, D), lambda i: (0, i, 0)),
            pl.BlockSpec((1, BR, D), lambda i: (1, i, 0)),
            pl.BlockSpec((D, D), lambda i: (0, 0)),
            pl.BlockSpec((D, D), lambda i: (0, 0)),
            pl.BlockSpec((1, D), lambda i: (0, 0)),
        ],
        out_specs=pl.BlockSpec((BR, D), lambda i: (i, 0)),
        out_shape=jax.ShapeDtypeStruct((N, D), jnp.float32),
    )(h, p, p, degp, degp, Ws, Wn, b.reshape(1, D))


def kernel(x, edge_index, W_self0, W_neigh0, b0, W_self1, W_neigh1, b1,
           W_self2, W_neigh2, b2):
    n_pad = EP - E
    pad_ids = jnp.arange(n_pad, dtype=jnp.int32)
    src = jnp.concatenate([edge_index[0], pad_ids % N]).reshape(EP // G, G)
    dst = jnp.concatenate([edge_index[1], N + pad_ids % (NP - N)]).reshape(
        EP // G, G)

    degp = _sc_deg(dst).reshape(NC, NP, D)

    params = [
        (W_self0, W_neigh0, b0, True),
        (W_self1, W_neigh1, b1, True),
        (W_self2, W_neigh2, b2, False),
    ]
    h = x
    for Ws, Wn, b, relu in params:
        p = _sc_agg(h, src, dst)
        h = _tc_combine(h, p, degp, Ws, Wn, b, relu)
    return h


# speedup vs baseline: 11.0402x; 1.0350x over previous
import functools

import jax
import jax.numpy as jnp
from jax import lax
from jax.experimental import pallas as pl
from jax.experimental.pallas import tpu as pltpu
from jax.experimental.pallas import tpu_sc as plsc

N = 10000
E = 320000
D = 128

NC = 2
NS = 16
NW = NC * NS
G = 128
NP = 10240
EP = NW * 80 * G
NCH = EP // NW // G
ROWS_PER_TILE = NP // NS

_MESH = plsc.VectorSubcoreMesh(core_axis_name="c", subcore_axis_name="s")


def _fill_vmem_2d(ref, rows, cols, value):
    vec = jnp.full((16,), value, jnp.float32)

    def row_body(r, _):
        def col_body(j, __):
            ref[r, pl.ds(j * 16, 16)] = vec
            return 0

        return lax.fori_loop(0, cols // 16, col_body, 0)

    lax.fori_loop(0, rows, row_body, 0)


@functools.partial(
    pl.kernel,
    out_type=jax.ShapeDtypeStruct((NC, NP, D), jnp.float32),
    mesh=_MESH,
    scratch_types=[
        pltpu.VMEM((NCH // 2, G), jnp.int32),
        pltpu.VMEM((NCH // 2, G), jnp.int32),
        pltpu.VMEM((G, D), jnp.float32),
        pltpu.VMEM((G, D), jnp.float32),
        pltpu.VMEM_SHARED((NP, D), jnp.float32),
        pltpu.SemaphoreType.DMA,
        pltpu.SemaphoreType.DMA,
    ],
)
def _sc_agg(h_hbm, src_hbm, dst_hbm, out_hbm, src_v, dst_v, rows0, rows1,
            acc_sh, sem0, sem1):
    c = lax.axis_index("c")
    s = lax.axis_index("s")
    wid = c * NS + s
    HCH = NCH // 2

    _fill_vmem_2d(rows0, G, D, 0.0)
    base_row = s * ROWS_PER_TILE
    for k in range(ROWS_PER_TILE // G):
        pltpu.sync_copy(rows0, acc_sh.at[pl.ds(base_row + k * G, G)])

    plsc.subcore_barrier()

    for b in range(2):
        pltpu.sync_copy(src_hbm.at[pl.ds(wid * NCH + b * HCH, HCH)], src_v)
        pltpu.sync_copy(dst_hbm.at[pl.ds(wid * NCH + b * HCH, HCH)], dst_v)
        pltpu.async_copy(h_hbm.at[src_v.at[0]], rows0, sem0)

        def body(jj, _):
            j = jj * 2
            pltpu.make_async_copy(h_hbm.at[src_v.at[0]], rows0, sem0).wait()
            pltpu.async_copy(h_hbm.at[src_v.at[j + 1]], rows1, sem1)
            pltpu.sync_copy(rows0, acc_sh.at[dst_v.at[j]], add=True)
            pltpu.make_async_copy(h_hbm.at[src_v.at[0]], rows1, sem1).wait()
            pltpu.async_copy(h_hbm.at[src_v.at[(j + 2) % HCH]], rows0, sem0)
            pltpu.sync_copy(rows1, acc_sh.at[dst_v.at[j + 1]], add=True)
            return 0

        lax.fori_loop(0, HCH // 2, body, 0)
        pltpu.make_async_copy(h_hbm.at[src_v.at[0]], rows0, sem0).wait()

    plsc.subcore_barrier()

    pltpu.sync_copy(
        acc_sh.at[pl.ds(base_row, ROWS_PER_TILE)],
        out_hbm.at[c, pl.ds(base_row, ROWS_PER_TILE)],
    )


@functools.partial(
    pl.kernel,
    out_type=jax.ShapeDtypeStruct((NC, NP, D), jnp.float32),
    mesh=_MESH,
    compiler_params=pltpu.CompilerParams(needs_layout_passes=False),
    scratch_types=[
        pltpu.VMEM((NCH, G), jnp.int32),
        pltpu.VMEM((NP,), jnp.float32),
        pltpu.VMEM((NS, ROWS_PER_TILE), jnp.float32),
        pltpu.VMEM((ROWS_PER_TILE,), jnp.float32),
        pltpu.VMEM((G, D), jnp.float32),
        pltpu.VMEM_SHARED((NS, NP), jnp.float32),
        pltpu.SemaphoreType.DMA,
    ],
)
def _sc_deg(dst_hbm, out_hbm, dst_v, hist, gath, red, brow, hists_sh, sem):
    c = lax.axis_index("c")
    s = lax.axis_index("s")
    wid = c * NS + s

    pltpu.sync_copy(dst_hbm.at[pl.ds(wid * NCH, NCH)], dst_v)

    zeros16 = jnp.zeros((16,), jnp.float32)

    def zb(i, _):
        hist[pl.ds(i * 16, 16)] = zeros16
        return 0

    lax.fori_loop(0, NP // 16, zb, 0)

    ones16 = jnp.ones((16,), jnp.float32)

    def hb(kk, _):
        idx = dst_v[kk // 8, pl.ds(lax.rem(kk, 8) * 16, 16)]
        plsc.addupdate_scatter(hist, [idx], ones16)
        return 0

    lax.fori_loop(0, NCH * 8, hb, 0)

    pltpu.sync_copy(hist, hists_sh.at[s])
    plsc.subcore_barrier()

    base = s * ROWS_PER_TILE
    pltpu.sync_copy(hists_sh.at[:, pl.ds(base, ROWS_PER_TILE)], gath)

    def rb(v, _):
        def rr(r, a):
            return a + gath[r, pl.ds(v * 16, 16)]

        red[pl.ds(v * 16, 16)] = lax.fori_loop(1, NS, rr,
                                               gath[0, pl.ds(v * 16, 16)])
        return 0

    lax.fori_loop(0, ROWS_PER_TILE // 16, rb, 0)

    lanes16 = lax.iota(jnp.int32, 16)
    for nb in range(ROWS_PER_TILE // G):
        def bb(n, _):
            val = plsc.load_gather(
                red, [jnp.full((16,), nb * G + n, jnp.int32)])
            rowid = jnp.full((16,), n, jnp.int32)

            def cb(q, _):
                plsc.store_scatter(brow, [rowid, lanes16 + q * 16], val)
                return 0

            return lax.fori_loop(0, D // 16, cb, 0)

        lax.fori_loop(0, G, bb, 0)
        pltpu.sync_copy(brow, out_hbm.at[c, pl.ds(base + nb * G, G)])


BR = 1000


def _combine_body(h_ref, p0_ref, p1_ref, d0_ref, d1_ref, ws_ref, wn_ref, b_ref,
                  o_ref, *, relu):
    deg = jnp.maximum(d0_ref[0, :, 0:1] + d1_ref[0, :, 0:1], 1.0)
    agg = (p0_ref[0] + p1_ref[0]) / deg
    out = (
        jnp.dot(h_ref[...], ws_ref[...], preferred_element_type=jnp.float32)
        + jnp.dot(agg, wn_ref[...], preferred_element_type=jnp.float32)
        + b_ref[...]
    )
    if relu:
        out = jnp.maximum(out, 0.0)
    o_ref[...] = out


def _tc_combine(h, p, degp, Ws, Wn, b, relu):
    return pl.pallas_call(
        functools.partial(_combine_body, relu=relu),
        grid=(N // BR,),
        in_specs=[
            pl.BlockSpec((BR, D), lambda i: (i, 0)),
            pl.BlockSpec((1, BR, D), lambda i: (0, i, 0)),
            pl.BlockSpec((1, BR, D), lambda i: (1, i, 0)),
            pl.BlockSpec((1, BR, D), lambda i: (0, i, 0)),
            pl.BlockSpec((1, BR, D), lambda i: (1, i, 0)),
            pl.BlockSpec((D, D), lambda i: (0, 0)),
            pl.BlockSpec((D, D), lambda i: (0, 0)),
            pl.BlockSpec((1, D), lambda i: (0, 0)),
        ],
        out_specs=pl.BlockSpec((BR, D), lambda i: (i, 0)),
        out_shape=jax.ShapeDtypeStruct((N, D), jnp.float32),
    )(h, p, p, degp, degp, Ws, Wn, b.reshape(1, D))


def kernel(x, edge_index, W_self0, W_neigh0, b0, W_self1, W_neigh1, b1,
           W_self2, W_neigh2, b2):
    n_pad = EP - E
    pad_ids = jnp.arange(n_pad, dtype=jnp.int32)
    src = jnp.concatenate([edge_index[0], pad_ids % N]).reshape(EP // G, G)
    dst = jnp.concatenate([edge_index[1], N + pad_ids % (NP - N)]).reshape(
        EP // G, G)

    degp = _sc_deg(dst)

    params = [
        (W_self0, W_neigh0, b0, True),
        (W_self1, W_neigh1, b1, True),
        (W_self2, W_neigh2, b2, False),
    ]
    h = x
    for Ws, Wn, b, relu in params:
        p = _sc_agg(h, src, dst)
        h = _tc_combine(h, p, degp, Ws, Wn, b, relu)
    return h


# speedup vs baseline: 11.1148x; 1.0068x over previous
import functools

import jax
import jax.numpy as jnp
from jax import lax
from jax.experimental import pallas as pl
from jax.experimental.pallas import tpu as pltpu
from jax.experimental.pallas import tpu_sc as plsc

N = 10000
E = 320000
D = 128

NC = 2
NS = 16
NW = NC * NS
G = 128
NP = 10240
EP = NW * 80 * G
NCH = EP // NW // G
ROWS_PER_TILE = NP // NS

_MESH = plsc.VectorSubcoreMesh(core_axis_name="c", subcore_axis_name="s")


def _fill_vmem_2d(ref, rows, cols, value):
    vec = jnp.full((16,), value, jnp.float32)

    def row_body(r, _):
        def col_body(j, __):
            ref[r, pl.ds(j * 16, 16)] = vec
            return 0

        return lax.fori_loop(0, cols // 16, col_body, 0)

    lax.fori_loop(0, rows, row_body, 0)


@functools.partial(
    pl.kernel,
    out_type=jax.ShapeDtypeStruct((NC, NP, D), jnp.float32),
    mesh=_MESH,
    scratch_types=[
        pltpu.VMEM((NCH // 2, G), jnp.int32),
        pltpu.VMEM((NCH // 2, G), jnp.int32),
        pltpu.VMEM((G, D), jnp.float32),
        pltpu.VMEM((G, D), jnp.float32),
        pltpu.VMEM_SHARED((NP, D), jnp.float32),
        pltpu.SemaphoreType.DMA,
        pltpu.SemaphoreType.DMA,
    ],
)
def _sc_agg(h_hbm, src_hbm, dst_hbm, out_hbm, src_v, dst_v, rows0, rows1,
            acc_sh, sem0, sem1):
    c = lax.axis_index("c")
    s = lax.axis_index("s")
    wid = c * NS + s
    HCH = NCH // 2

    _fill_vmem_2d(rows0, G, D, 0.0)
    base_row = s * ROWS_PER_TILE
    for k in range(ROWS_PER_TILE // G):
        pltpu.async_copy(rows0, acc_sh.at[pl.ds(base_row + k * G, G)], sem0)
    pltpu.sync_copy(src_hbm.at[pl.ds(wid * NCH, HCH)], src_v)
    pltpu.sync_copy(dst_hbm.at[pl.ds(wid * NCH, HCH)], dst_v)
    for k in range(ROWS_PER_TILE // G):
        pltpu.make_async_copy(rows0, acc_sh.at[pl.ds(base_row, G)],
                              sem0).wait()

    plsc.subcore_barrier()

    for b in range(2):
        if b > 0:
            pltpu.sync_copy(src_hbm.at[pl.ds(wid * NCH + b * HCH, HCH)],
                            src_v)
            pltpu.sync_copy(dst_hbm.at[pl.ds(wid * NCH + b * HCH, HCH)],
                            dst_v)
        pltpu.async_copy(h_hbm.at[src_v.at[0]], rows0, sem0)

        def body(jj, _):
            j = jj * 2
            pltpu.make_async_copy(h_hbm.at[src_v.at[0]], rows0, sem0).wait()
            pltpu.async_copy(h_hbm.at[src_v.at[j + 1]], rows1, sem1)
            pltpu.sync_copy(rows0, acc_sh.at[dst_v.at[j]], add=True)
            pltpu.make_async_copy(h_hbm.at[src_v.at[0]], rows1, sem1).wait()
            pltpu.async_copy(h_hbm.at[src_v.at[(j + 2) % HCH]], rows0, sem0)
            pltpu.sync_copy(rows1, acc_sh.at[dst_v.at[j + 1]], add=True)
            return 0

        lax.fori_loop(0, HCH // 2, body, 0)
        pltpu.make_async_copy(h_hbm.at[src_v.at[0]], rows0, sem0).wait()

    plsc.subcore_barrier()

    pltpu.sync_copy(
        acc_sh.at[pl.ds(base_row, ROWS_PER_TILE)],
        out_hbm.at[c, pl.ds(base_row, ROWS_PER_TILE)],
    )


@functools.partial(
    pl.kernel,
    out_type=jax.ShapeDtypeStruct((NC, NP, D), jnp.float32),
    mesh=_MESH,
    compiler_params=pltpu.CompilerParams(needs_layout_passes=False),
    scratch_types=[
        pltpu.VMEM((NCH, G), jnp.int32),
        pltpu.VMEM((NP,), jnp.float32),
        pltpu.VMEM((NS, ROWS_PER_TILE), jnp.float32),
        pltpu.VMEM((ROWS_PER_TILE,), jnp.float32),
        pltpu.VMEM((G, D), jnp.float32),
        pltpu.VMEM_SHARED((NS, NP), jnp.float32),
        pltpu.SemaphoreType.DMA,
    ],
)
def _sc_deg(dst_hbm, out_hbm, dst_v, hist, gath, red, brow, hists_sh, sem):
    c = lax.axis_index("c")
    s = lax.axis_index("s")
    wid = c * NS + s

    pltpu.sync_copy(dst_hbm.at[pl.ds(wid * NCH, NCH)], dst_v)

    zeros16 = jnp.zeros((16,), jnp.float32)

    def zb(i, _):
        hist[pl.ds(i * 16, 16)] = zeros16
        return 0

    lax.fori_loop(0, NP // 16, zb, 0)

    ones16 = jnp.ones((16,), jnp.float32)

    def hb(kk, _):
        idx = dst_v[kk // 8, pl.ds(lax.rem(kk, 8) * 16, 16)]
        plsc.addupdate_scatter(hist, [idx], ones16)
        return 0

    lax.fori_loop(0, NCH * 8, hb, 0)

    pltpu.sync_copy(hist, hists_sh.at[s])
    plsc.subcore_barrier()

    base = s * ROWS_PER_TILE
    pltpu.sync_copy(hists_sh.at[:, pl.ds(base, ROWS_PER_TILE)], gath)

    def rb(v, _):
        def rr(r, a):
            return a + gath[r, pl.ds(v * 16, 16)]

        red[pl.ds(v * 16, 16)] = lax.fori_loop(1, NS, rr,
                                               gath[0, pl.ds(v * 16, 16)])
        return 0

    lax.fori_loop(0, ROWS_PER_TILE // 16, rb, 0)

    lanes16 = lax.iota(jnp.int32, 16)
    for nb in range(ROWS_PER_TILE // G):
        def bb(n, _):
            val = plsc.load_gather(
                red, [jnp.full((16,), nb * G + n, jnp.int32)])
            rowid = jnp.full((16,), n, jnp.int32)

            def cb(q, _):
                plsc.store_scatter(brow, [rowid, lanes16 + q * 16], val)
                return 0

            return lax.fori_loop(0, D // 16, cb, 0)

        lax.fori_loop(0, G, bb, 0)
        pltpu.sync_copy(brow, out_hbm.at[c, pl.ds(base + nb * G, G)])


BR = 1000


def _combine_body(h_ref, p0_ref, p1_ref, d0_ref, d1_ref, ws_ref, wn_ref, b_ref,
                  o_ref, *, relu):
    deg = jnp.maximum(d0_ref[0, :, 0:1] + d1_ref[0, :, 0:1], 1.0)
    agg = (p0_ref[0] + p1_ref[0]) / deg
    out = (
        jnp.dot(h_ref[...], ws_ref[...], preferred_element_type=jnp.float32)
        + jnp.dot(agg, wn_ref[...], preferred_element_type=jnp.float32)
        + b_ref[...]
    )
    if relu:
        out = jnp.maximum(out, 0.0)
    o_ref[...] = out


def _tc_combine(h, p, degp, Ws, Wn, b, relu):
    return pl.pallas_call(
        functools.partial(_combine_body, relu=relu),
        grid=(N // BR,),
        in_specs=[
            pl.BlockSpec((BR, D), lambda i: (i, 0)),
            pl.BlockSpec((1, BR, D), lambda i: (0, i, 0)),
            pl.BlockSpec((1, BR, D), lambda i: (1, i, 0)),
            pl.BlockSpec((1, BR, D), lambda i: (0, i, 0)),
            pl.BlockSpec((1, BR, D), lambda i: (1, i, 0)),
            pl.BlockSpec((D, D), lambda i: (0, 0)),
            pl.BlockSpec((D, D), lambda i: (0, 0)),
            pl.BlockSpec((1, D), lambda i: (0, 0)),
        ],
        out_specs=pl.BlockSpec((BR, D), lambda i: (i, 0)),
        out_shape=jax.ShapeDtypeStruct((N, D), jnp.float32),
    )(h, p, p, degp, degp, Ws, Wn, b.reshape(1, D))


def kernel(x, edge_index, W_self0, W_neigh0, b0, W_self1, W_neigh1, b1,
           W_self2, W_neigh2, b2):
    n_pad = EP - E
    pad_ids = jnp.arange(n_pad, dtype=jnp.int32)
    src = jnp.concatenate([edge_index[0], pad_ids % N]).reshape(EP // G, G)
    dst = jnp.concatenate([edge_index[1], N + pad_ids % (NP - N)]).reshape(
        EP // G, G)

    degp = _sc_deg(dst)

    params = [
        (W_self0, W_neigh0, b0, True),
        (W_self1, W_neigh1, b1, True),
        (W_self2, W_neigh2, b2, False),
    ]
    h = x
    for Ws, Wn, b, relu in params:
        p = _sc_agg(h, src, dst)
        h = _tc_combine(h, p, degp, Ws, Wn, b, relu)
    return h
